# baseline (speedup 1.0000x reference)
import jax
import jax.numpy as jnp
from jax import lax
from jax.experimental import pallas as pl
from jax.experimental.pallas import tpu as pltpu
from jax.experimental.pallas import tpu_sc as plsc

N = 10000
E = 320000
D_IN = 128
D_H = 64
D_OUT = 128

NUM_SUBCORES = 16
CHUNK = 128
NBUF = 4
SLACK = 2
NCH = 160
HALF = NCH // 2
TOTAL_CH = NUM_SUBCORES * NCH
E_PAD = TOTAL_CH * CHUNK
N_PAD = 10112
RPT = N_PAD // NUM_SUBCORES
CNT_W = 16
L = 16


def _zero_vmem_2d(ref, rows, cols):
    z = jnp.zeros((L,), jnp.float32)

    def zrow(i, carry):
        for k in range(cols // L):
            ref[i, pl.ds(k * L, L)] = z
        return carry

    lax.fori_loop(0, rows, zrow, 0)


def _make_sc_segsum(with_counts):
    mesh = plsc.VectorSubcoreMesh(core_axis_name="c", subcore_axis_name="s",
                                  num_cores=2, num_subcores=NUM_SUBCORES)
    out_type = [jax.ShapeDtypeStruct((N_PAD, D_H), jnp.float32)]
    scratch = [
        pltpu.VMEM((HALF, CHUNK), jnp.int32),
        pltpu.VMEM((HALF, CHUNK), jnp.int32),
        pltpu.VMEM((NBUF, CHUNK, D_H), jnp.float32),
        pltpu.VMEM_SHARED((N_PAD, D_H), jnp.float32),
    ] + [pltpu.SemaphoreType.DMA] * (2 * NBUF)
    if with_counts:
        out_type.append(jax.ShapeDtypeStruct((N_PAD, CNT_W), jnp.float32))
        scratch += [
            pltpu.VMEM((CHUNK, CNT_W), jnp.float32),
            pltpu.VMEM((CHUNK, CNT_W), jnp.float32),
            pltpu.VMEM_SHARED((N_PAD, CNT_W), jnp.float32),
            pltpu.SemaphoreType.DMA,
        ]

    def body(p_hbm, srcs_hbm, dsts_hbm, *rest):
        if with_counts:
            (ones_hbm, s_out, c_out, src_v, dst_v, rows_v, acc,
             *sems, ones_v, zc_v, cacc, csem) = rest
        else:
            s_out, src_v, dst_v, rows_v, acc, *sems = rest
        gsem = sems[:NBUF]
        ssem = sems[NBUF:2 * NBUF]
        c = lax.axis_index("c")
        s = lax.axis_index("s")
        row0 = s * RPT

        def stage(base):
            pltpu.sync_copy(srcs_hbm.at[pl.ds(s * NCH + base, HALF)], src_v)
            pltpu.sync_copy(dsts_hbm.at[pl.ds(s * NCH + base, HALF)], dst_v)

        def phase():
            for b in range(NBUF):
                pltpu.async_copy(p_hbm.at[src_v.at[b]], rows_v.at[b],
                                 gsem[b])

            def group_body(g, carry):
                for b in range(NBUF):
                    j = g * NBUF + b
                    pltpu.make_async_copy(
                        p_hbm.at[src_v.at[j]], rows_v.at[b], gsem[b]).wait()
                    pltpu.async_copy(rows_v.at[b], acc.at[dst_v.at[j]],
                                     ssem[b], add=True)
                    if with_counts:
                        pltpu.async_copy(ones_v, cacc.at[dst_v.at[j]],
                                         csem, add=True)

                        @pl.when(j >= 2 * NBUF)
                        def _():
                            pltpu.make_async_copy(
                                ones_v, cacc.at[dst_v.at[0]], csem).wait()
                    br = (b + SLACK) % NBUF
                    jd = j - SLACK
                    jr = jd + NBUF

                    @pl.when(jnp.logical_and(jd >= 0, jr < HALF))
                    def _():
                        pltpu.make_async_copy(
                            rows_v.at[br], acc.at[dst_v.at[jd]],
                            ssem[br]).wait()
                        pltpu.async_copy(
                            p_hbm.at[src_v.at[jr]], rows_v.at[br], gsem[br])
                return carry

            lax.fori_loop(0, HALF // NBUF, group_body, 0)
            for b in range(NBUF):
                pltpu.make_async_copy(
                    rows_v.at[b], acc.at[dst_v.at[0]], ssem[b]).wait()

        def work():
            r0 = rows_v.at[0]
            _zero_vmem_2d(r0, CHUNK, D_H)
            for k in range(RPT // CHUNK):
                pltpu.sync_copy(r0, acc.at[pl.ds(row0 + k * CHUNK, CHUNK)])
            rem = RPT % CHUNK
            if rem:
                pltpu.sync_copy(
                    r0.at[pl.ds(0, rem)],
                    acc.at[pl.ds(row0 + (RPT // CHUNK) * CHUNK, rem)])
            if with_counts:
                _zero_vmem_2d(zc_v, CHUNK, CNT_W)
                for k in range(RPT // CHUNK):
                    pltpu.sync_copy(
                        zc_v, cacc.at[pl.ds(row0 + k * CHUNK, CHUNK)])
                if rem:
                    pltpu.sync_copy(
                        zc_v.at[pl.ds(0, rem)],
                        cacc.at[pl.ds(row0 + (RPT // CHUNK) * CHUNK, rem)])
                pltpu.sync_copy(ones_hbm, ones_v)
            stage(0)
            plsc.subcore_barrier()
            phase()
            stage(HALF)
            phase()
            if with_counts:
                def cdrain(j, carry):
                    pltpu.make_async_copy(
                        ones_v, cacc.at[dst_v.at[0]], csem).wait()
                    return carry
                lax.fori_loop(0, 4 * NBUF, cdrain, 0)
            plsc.subcore_barrier()
            pltpu.sync_copy(acc.at[pl.ds(row0, RPT)],
                            s_out.at[pl.ds(row0, RPT)])
            if with_counts:
                pltpu.sync_copy(cacc.at[pl.ds(row0, RPT)],
                                c_out.at[pl.ds(row0, RPT)])

        pl.when(c == 0)(work)

    return pl.kernel(body, out_type=tuple(out_type), mesh=mesh,
                     scratch_types=tuple(scratch),
                     compiler_params=pltpu.CompilerParams(
                         use_tc_tiling_on_sc=False))


def _tc_proj(h, Wl, Wr, b2d):
    n = h.shape[0]
    dout = Wl.shape[1]

    def body(h_ref, wl_ref, wr_ref, b_ref, p_ref, q_ref):
        hv = h_ref[...]
        p_ref[...] = jnp.dot(hv, wl_ref[...],
                             preferred_element_type=jnp.float32)
        q_ref[...] = jnp.dot(hv, wr_ref[...],
                             preferred_element_type=jnp.float32) + b_ref[...]

    return pl.pallas_call(
        body,
        out_shape=(jax.ShapeDtypeStruct((n, dout), jnp.float32),
                   jax.ShapeDtypeStruct((n, dout), jnp.float32)),
    )(h, Wl, Wr, b2d)


def _tc_combine(s_sum, cnts, q, g2, be2, m2, v2):
    def body(s_ref, c_ref, q_ref, g_ref, be_ref, m_ref, v_ref, o_ref):
        cnt = c_ref[:N, 0:1]
        recip = 1.0 / jnp.maximum(cnt, 1.0)
        t = s_ref[:N, :] * recip + q_ref[...]
        scale = g_ref[...] * lax.rsqrt(v_ref[...] + 1e-5)
        o_ref[...] = jnp.maximum((t - m_ref[...]) * scale + be_ref[...], 0.0)

    return pl.pallas_call(
        body, out_shape=jax.ShapeDtypeStruct((N, D_H), jnp.float32),
    )(s_sum, cnts, q, g2, be2, m2, v2)


def _tc_final(s_sum, cnts, h, Wl, Wr, b2d):
    def body(s_ref, c_ref, h_ref, wl_ref, wr_ref, b_ref, o_ref):
        cnt = c_ref[:N, 0:1]
        agg = s_ref[:N, :] * (1.0 / jnp.maximum(cnt, 1.0))
        o_ref[...] = (
            jnp.dot(agg, wl_ref[...], preferred_element_type=jnp.float32)
            + jnp.dot(h_ref[...], wr_ref[...],
                      preferred_element_type=jnp.float32)
            + b_ref[...])

    return pl.pallas_call(
        body, out_shape=jax.ShapeDtypeStruct((N, D_OUT), jnp.float32),
    )(s_sum, cnts, h, Wl, Wr, b2d)


def kernel(x, edge_index, Wl0, Wr0, b0, g0, be0, m0, v0,
           Wl1, Wr1, b1, g1, be1, m1, v1, Wl2, Wr2, b2):
    src = edge_index[0]
    dst = edge_index[1]
    pad = E_PAD - E
    srcs = jnp.concatenate(
        [src, jnp.zeros((pad,), jnp.int32)]).reshape(TOTAL_CH, CHUNK)
    dsts = jnp.concatenate(
        [dst, jnp.full((pad,), N, jnp.int32)]).reshape(TOTAL_CH, CHUNK)
    ones = jnp.ones((CHUNK, CNT_W), jnp.float32)

    seg_cnt = _make_sc_segsum(True)
    seg = _make_sc_segsum(False)

    r2 = lambda a: a.reshape(1, -1)

    p0, q0 = _tc_proj(x, Wl0, Wr0, r2(b0))
    s0, cnt = seg_cnt(p0, srcs, dsts, ones)
    h1 = _tc_combine(s0, cnt, q0, r2(g0), r2(be0), r2(m0), r2(v0))

    p1, q1 = _tc_proj(h1, Wl1, Wr1, r2(b1))
    outs = seg(p1, srcs, dsts)
    s1 = outs[0] if isinstance(outs, (tuple, list)) else outs
    h2 = _tc_combine(s1, cnt, q1, r2(g1), r2(be1), r2(m1), r2(v1))

    outs = seg(h2, srcs, dsts)
    s2 = outs[0] if isinstance(outs, (tuple, list)) else outs
    return _tc_final(s2, cnt, h2, Wl2, Wr2, r2(b2))


# speedup vs baseline: 1.3600x; 1.3600x over previous
import jax
import jax.numpy as jnp
from jax import lax
from jax.experimental import pallas as pl
from jax.experimental.pallas import tpu as pltpu
from jax.experimental.pallas import tpu_sc as plsc

N = 10000
E = 320000
D_IN = 128
D_H = 64
D_OUT = 128

NUM_CORES = 2
NUM_SUBCORES = 16
CHUNK = 128
NBUF = 4
SLACK = 2
NCH0 = 148
NCH1 = 12
TOTAL_CH = NUM_SUBCORES * (NCH0 + NCH1)
E_PAD = TOTAL_CH * CHUNK
N_PAD = 10112
RPT = N_PAD // NUM_SUBCORES
CNT_W = 16
L = 16


def _zero_vmem_2d(ref, rows, cols):
    z = jnp.zeros((L,), jnp.float32)

    def zrow(i, carry):
        for k in range(cols // L):
            ref[i, pl.ds(k * L, L)] = z
        return carry

    lax.fori_loop(0, rows, zrow, 0)


def _make_sc_segsum(with_counts):
    mesh = plsc.VectorSubcoreMesh(core_axis_name="c", subcore_axis_name="s",
                                  num_cores=NUM_CORES,
                                  num_subcores=NUM_SUBCORES)
    out_type = [jax.ShapeDtypeStruct((NUM_CORES, N_PAD, D_H), jnp.float32)]
    scratch = [
        pltpu.VMEM((NCH0, CHUNK), jnp.int32),
        pltpu.VMEM((NCH0, CHUNK), jnp.int32),
        pltpu.VMEM((NBUF, CHUNK, D_H), jnp.float32),
        pltpu.VMEM_SHARED((N_PAD, D_H), jnp.float32),
    ] + [pltpu.SemaphoreType.DMA] * (2 * NBUF)
    if with_counts:
        out_type.append(
            jax.ShapeDtypeStruct((NUM_CORES, N_PAD, CNT_W), jnp.float32))
        scratch += [
            pltpu.VMEM((CHUNK, CNT_W), jnp.float32),
            pltpu.VMEM((CHUNK, CNT_W), jnp.float32),
            pltpu.VMEM_SHARED((N_PAD, CNT_W), jnp.float32),
            pltpu.SemaphoreType.DMA,
        ]

    def body(p_hbm, srcs_hbm, dsts_hbm, *rest):
        if with_counts:
            (ones_hbm, s_out, c_out, src_v, dst_v, rows_v, acc,
             *sems, ones_v, zc_v, cacc, csem) = rest
        else:
            s_out, src_v, dst_v, rows_v, acc, *sems = rest
        gsem = sems[:NBUF]
        ssem = sems[NBUF:2 * NBUF]
        c = lax.axis_index("c")
        s = lax.axis_index("s")
        row0 = s * RPT
        n = jnp.where(c == 0, NCH0, NCH1)

        r0 = rows_v.at[0]
        _zero_vmem_2d(r0, CHUNK, D_H)
        for k in range(RPT // CHUNK):
            pltpu.sync_copy(r0, acc.at[pl.ds(row0 + k * CHUNK, CHUNK)])
        rem = RPT % CHUNK
        if rem:
            pltpu.sync_copy(r0.at[pl.ds(0, rem)],
                            acc.at[pl.ds(row0 + (RPT // CHUNK) * CHUNK, rem)])
        if with_counts:
            _zero_vmem_2d(zc_v, CHUNK, CNT_W)
            for k in range(RPT // CHUNK):
                pltpu.sync_copy(zc_v,
                                cacc.at[pl.ds(row0 + k * CHUNK, CHUNK)])
            if rem:
                pltpu.sync_copy(
                    zc_v.at[pl.ds(0, rem)],
                    cacc.at[pl.ds(row0 + (RPT // CHUNK) * CHUNK, rem)])
            pltpu.sync_copy(ones_hbm, ones_v)

        @pl.when(c == 0)
        def _():
            pltpu.sync_copy(srcs_hbm.at[pl.ds(s * NCH0, NCH0)], src_v)
            pltpu.sync_copy(dsts_hbm.at[pl.ds(s * NCH0, NCH0)], dst_v)

        @pl.when(c == 1)
        def _():
            base = NUM_SUBCORES * NCH0 + s * NCH1
            pltpu.sync_copy(srcs_hbm.at[pl.ds(base, NCH1)],
                            src_v.at[pl.ds(0, NCH1)])
            pltpu.sync_copy(dsts_hbm.at[pl.ds(base, NCH1)],
                            dst_v.at[pl.ds(0, NCH1)])

        plsc.subcore_barrier()

        for b in range(NBUF):
            pltpu.async_copy(p_hbm.at[src_v.at[b]], rows_v.at[b], gsem[b])

        def group_body(g, carry):
            for b in range(NBUF):
                j = g * NBUF + b
                pltpu.make_async_copy(
                    p_hbm.at[src_v.at[j]], rows_v.at[b], gsem[b]).wait()
                pltpu.async_copy(rows_v.at[b], acc.at[dst_v.at[j]],
                                 ssem[b], add=True)
                if with_counts:
                    pltpu.async_copy(ones_v, cacc.at[dst_v.at[j]],
                                     csem, add=True)

                    @pl.when(j >= 2 * NBUF)
                    def _():
                        pltpu.make_async_copy(
                            ones_v, cacc.at[dst_v.at[0]], csem).wait()
                br = (b + SLACK) % NBUF
                jd = j - SLACK
                jr = jd + NBUF

                @pl.when(jnp.logical_and(jd >= 0, jr < n))
                def _():
                    pltpu.make_async_copy(
                        rows_v.at[br], acc.at[dst_v.at[jd]],
                        ssem[br]).wait()
                    pltpu.async_copy(
                        p_hbm.at[src_v.at[jr]], rows_v.at[br], gsem[br])
            return carry

        lax.fori_loop(0, n // NBUF, group_body, 0)
        for b in range(NBUF):
            pltpu.make_async_copy(
                rows_v.at[b], acc.at[dst_v.at[0]], ssem[b]).wait()
        if with_counts:
            def cdrain(j, carry):
                pltpu.make_async_copy(
                    ones_v, cacc.at[dst_v.at[0]], csem).wait()
                return carry
            lax.fori_loop(0, 2 * NBUF, cdrain, 0)
        plsc.subcore_barrier()
        pltpu.sync_copy(acc.at[pl.ds(row0, RPT)],
                        s_out.at[c, pl.ds(row0, RPT)])
        if with_counts:
            pltpu.sync_copy(cacc.at[pl.ds(row0, RPT)],
                            c_out.at[c, pl.ds(row0, RPT)])

    return pl.kernel(body, out_type=tuple(out_type), mesh=mesh,
                     scratch_types=tuple(scratch),
                     compiler_params=pltpu.CompilerParams(
                         use_tc_tiling_on_sc=False))


def _tc_proj(h, Wl, Wr, b2d):
    n = h.shape[0]
    dout = Wl.shape[1]

    def body(h_ref, wl_ref, wr_ref, b_ref, p_ref, q_ref):
        hv = h_ref[...]
        p_ref[...] = jnp.dot(hv, wl_ref[...],
                             preferred_element_type=jnp.float32)
        q_ref[...] = jnp.dot(hv, wr_ref[...],
                             preferred_element_type=jnp.float32) + b_ref[...]

    return pl.pallas_call(
        body,
        out_shape=(jax.ShapeDtypeStruct((n, dout), jnp.float32),
                   jax.ShapeDtypeStruct((n, dout), jnp.float32)),
    )(h, Wl, Wr, b2d)


def _tc_combine(s_parts, cnt_parts, q, g2, be2, m2, v2):
    def body(s_ref, c_ref, q_ref, g_ref, be_ref, m_ref, v_ref, o_ref):
        ssum = s_ref[0, :N, :] + s_ref[1, :N, :]
        cnt = c_ref[0, :N, 0:1] + c_ref[1, :N, 0:1]
        recip = 1.0 / jnp.maximum(cnt, 1.0)
        t = ssum * recip + q_ref[...]
        scale = g_ref[...] * lax.rsqrt(v_ref[...] + 1e-5)
        o_ref[...] = jnp.maximum((t - m_ref[...]) * scale + be_ref[...], 0.0)

    return pl.pallas_call(
        body, out_shape=jax.ShapeDtypeStruct((N, D_H), jnp.float32),
    )(s_parts, cnt_parts, q, g2, be2, m2, v2)


def _tc_final(s_parts, cnt_parts, h, Wl, Wr, b2d):
    def body(s_ref, c_ref, h_ref, wl_ref, wr_ref, b_ref, o_ref):
        ssum = s_ref[0, :N, :] + s_ref[1, :N, :]
        cnt = c_ref[0, :N, 0:1] + c_ref[1, :N, 0:1]
        agg = ssum * (1.0 / jnp.maximum(cnt, 1.0))
        o_ref[...] = (
            jnp.dot(agg, wl_ref[...], preferred_element_type=jnp.float32)
            + jnp.dot(h_ref[...], wr_ref[...],
                      preferred_element_type=jnp.float32)
            + b_ref[...])

    return pl.pallas_call(
        body, out_shape=jax.ShapeDtypeStruct((N, D_OUT), jnp.float32),
    )(s_parts, cnt_parts, h, Wl, Wr, b2d)


def kernel(x, edge_index, Wl0, Wr0, b0, g0, be0, m0, v0,
           Wl1, Wr1, b1, g1, be1, m1, v1, Wl2, Wr2, b2):
    src = edge_index[0]
    dst = edge_index[1]
    pad = E_PAD - E
    srcs = jnp.concatenate(
        [src, jnp.zeros((pad,), jnp.int32)]).reshape(TOTAL_CH, CHUNK)
    dsts = jnp.concatenate(
        [dst, jnp.full((pad,), N, jnp.int32)]).reshape(TOTAL_CH, CHUNK)
    ones = jnp.ones((CHUNK, CNT_W), jnp.float32)

    seg_cnt = _make_sc_segsum(True)
    seg = _make_sc_segsum(False)

    r2 = lambda a: a.reshape(1, -1)

    p0, q0 = _tc_proj(x, Wl0, Wr0, r2(b0))
    s0, cnt = seg_cnt(p0, srcs, dsts, ones)
    h1 = _tc_combine(s0, cnt, q0, r2(g0), r2(be0), r2(m0), r2(v0))

    p1, q1 = _tc_proj(h1, Wl1, Wr1, r2(b1))
    outs = seg(p1, srcs, dsts)
    s1 = outs[0] if isinstance(outs, (tuple, list)) else outs
    h2 = _tc_combine(s1, cnt, q1, r2(g1), r2(be1), r2(m1), r2(v1))

    outs = seg(h2, srcs, dsts)
    s2 = outs[0] if isinstance(outs, (tuple, list)) else outs
    return _tc_final(s2, cnt, h2, Wl2, Wr2, r2(b2))


# speedup vs baseline: 1.3683x; 1.0061x over previous
import jax
import jax.numpy as jnp
from jax import lax
from jax.experimental import pallas as pl
from jax.experimental.pallas import tpu as pltpu
from jax.experimental.pallas import tpu_sc as plsc

N = 10000
E = 320000
D_IN = 128
D_H = 64
D_OUT = 128

NUM_CORES = 2
NUM_SUBCORES = 16
CHUNK = 128
NBUF = 4
SLACK = 2
NCH0 = 152
NCH1 = 8
TOTAL_CH = NUM_SUBCORES * (NCH0 + NCH1)
E_PAD = TOTAL_CH * CHUNK
N_PAD = 10112
RPT = N_PAD // NUM_SUBCORES
CNT_W = 16
L = 16


def _zero_vmem_2d(ref, rows, cols):
    z = jnp.zeros((L,), jnp.float32)

    def zrow(i, carry):
        for k in range(cols // L):
            ref[i, pl.ds(k * L, L)] = z
        return carry

    lax.fori_loop(0, rows, zrow, 0)


def _make_sc_segsum(with_counts):
    mesh = plsc.VectorSubcoreMesh(core_axis_name="c", subcore_axis_name="s",
                                  num_cores=NUM_CORES,
                                  num_subcores=NUM_SUBCORES)
    out_type = [jax.ShapeDtypeStruct((NUM_CORES, N_PAD, D_H), jnp.float32)]
    scratch = [
        pltpu.VMEM((NCH0, CHUNK), jnp.int32),
        pltpu.VMEM((NCH0, CHUNK), jnp.int32),
        pltpu.VMEM((NBUF, CHUNK, D_H), jnp.float32),
        pltpu.VMEM_SHARED((N_PAD, D_H), jnp.float32),
    ] + [pltpu.SemaphoreType.DMA] * (2 * NBUF)
    if with_counts:
        out_type.append(
            jax.ShapeDtypeStruct((NUM_CORES, N_PAD, CNT_W), jnp.float32))
        scratch += [
            pltpu.VMEM((CHUNK, CNT_W), jnp.float32),
            pltpu.VMEM((CHUNK, CNT_W), jnp.float32),
            pltpu.VMEM_SHARED((N_PAD, CNT_W), jnp.float32),
            pltpu.SemaphoreType.DMA,
        ]

    def body(p_hbm, srcs_hbm, dsts_hbm, *rest):
        if with_counts:
            (ones_hbm, s_out, c_out, src_v, dst_v, rows_v, acc,
             *sems, ones_v, zc_v, cacc, csem) = rest
        else:
            s_out, src_v, dst_v, rows_v, acc, *sems = rest
        gsem = sems[:NBUF]
        ssem = sems[NBUF:2 * NBUF]
        c = lax.axis_index("c")
        s = lax.axis_index("s")
        row0 = s * RPT
        n = jnp.where(c == 0, NCH0, NCH1)

        r0 = rows_v.at[0]
        _zero_vmem_2d(r0, CHUNK, D_H)
        for k in range(RPT // CHUNK):
            pltpu.sync_copy(r0, acc.at[pl.ds(row0 + k * CHUNK, CHUNK)])
        rem = RPT % CHUNK
        if rem:
            pltpu.sync_copy(r0.at[pl.ds(0, rem)],
                            acc.at[pl.ds(row0 + (RPT // CHUNK) * CHUNK, rem)])
        if with_counts:
            _zero_vmem_2d(zc_v, CHUNK, CNT_W)
            for k in range(RPT // CHUNK):
                pltpu.sync_copy(zc_v,
                                cacc.at[pl.ds(row0 + k * CHUNK, CHUNK)])
            if rem:
                pltpu.sync_copy(
                    zc_v.at[pl.ds(0, rem)],
                    cacc.at[pl.ds(row0 + (RPT // CHUNK) * CHUNK, rem)])
            pltpu.sync_copy(ones_hbm, ones_v)

        @pl.when(c == 0)
        def _():
            pltpu.sync_copy(srcs_hbm.at[pl.ds(s * NCH0, NCH0)], src_v)
            pltpu.sync_copy(dsts_hbm.at[pl.ds(s * NCH0, NCH0)], dst_v)

        @pl.when(c == 1)
        def _():
            base = NUM_SUBCORES * NCH0 + s * NCH1
            pltpu.sync_copy(srcs_hbm.at[pl.ds(base, NCH1)],
                            src_v.at[pl.ds(0, NCH1)])
            pltpu.sync_copy(dsts_hbm.at[pl.ds(base, NCH1)],
                            dst_v.at[pl.ds(0, NCH1)])

        plsc.subcore_barrier()

        for b in range(NBUF):
            pltpu.async_copy(p_hbm.at[src_v.at[b]], rows_v.at[b], gsem[b])

        def group_body(g, carry):
            for b in range(NBUF):
                j = g * NBUF + b
                pltpu.make_async_copy(
                    p_hbm.at[src_v.at[j]], rows_v.at[b], gsem[b]).wait()
                pltpu.async_copy(rows_v.at[b], acc.at[dst_v.at[j]],
                                 ssem[b], add=True)
                if with_counts:
                    pltpu.async_copy(ones_v, cacc.at[dst_v.at[j]],
                                     csem, add=True)

                    @pl.when(j >= 2 * NBUF)
                    def _():
                        pltpu.make_async_copy(
                            ones_v, cacc.at[dst_v.at[0]], csem).wait()
                br = (b + SLACK) % NBUF
                jd = j - SLACK
                jr = jd + NBUF

                @pl.when(jnp.logical_and(jd >= 0, jr < n))
                def _():
                    pltpu.make_async_copy(
                        rows_v.at[br], acc.at[dst_v.at[jd]],
                        ssem[br]).wait()
                    pltpu.async_copy(
                        p_hbm.at[src_v.at[jr]], rows_v.at[br], gsem[br])
            return carry

        lax.fori_loop(0, n // NBUF, group_body, 0)
        for b in range(NBUF):
            pltpu.make_async_copy(
                rows_v.at[b], acc.at[dst_v.at[0]], ssem[b]).wait()
        if with_counts:
            def cdrain(j, carry):
                pltpu.make_async_copy(
                    ones_v, cacc.at[dst_v.at[0]], csem).wait()
                return carry
            lax.fori_loop(0, 2 * NBUF, cdrain, 0)
        plsc.subcore_barrier()
        pltpu.sync_copy(acc.at[pl.ds(row0, RPT)],
                        s_out.at[c, pl.ds(row0, RPT)])
        if with_counts:
            pltpu.sync_copy(cacc.at[pl.ds(row0, RPT)],
                            c_out.at[c, pl.ds(row0, RPT)])

    return pl.kernel(body, out_type=tuple(out_type), mesh=mesh,
                     scratch_types=tuple(scratch),
                     compiler_params=pltpu.CompilerParams(
                         use_tc_tiling_on_sc=False))


def _tc_proj(h, Wl, Wr, b2d):
    n = h.shape[0]
    dout = Wl.shape[1]

    def body(h_ref, wl_ref, wr_ref, b_ref, p_ref, q_ref):
        hv = h_ref[...]
        p_ref[...] = jnp.dot(hv, wl_ref[...],
                             preferred_element_type=jnp.float32)
        q_ref[...] = jnp.dot(hv, wr_ref[...],
                             preferred_element_type=jnp.float32) + b_ref[...]

    return pl.pallas_call(
        body,
        out_shape=(jax.ShapeDtypeStruct((n, dout), jnp.float32),
                   jax.ShapeDtypeStruct((n, dout), jnp.float32)),
    )(h, Wl, Wr, b2d)


def _tc_combine(s_parts, cnt_parts, q, g2, be2, m2, v2):
    def body(s_ref, c_ref, q_ref, g_ref, be_ref, m_ref, v_ref, o_ref):
        ssum = s_ref[0, :N, :] + s_ref[1, :N, :]
        cnt = c_ref[0, :N, 0:1] + c_ref[1, :N, 0:1]
        recip = 1.0 / jnp.maximum(cnt, 1.0)
        t = ssum * recip + q_ref[...]
        scale = g_ref[...] * lax.rsqrt(v_ref[...] + 1e-5)
        o_ref[...] = jnp.maximum((t - m_ref[...]) * scale + be_ref[...], 0.0)

    return pl.pallas_call(
        body, out_shape=jax.ShapeDtypeStruct((N, D_H), jnp.float32),
    )(s_parts, cnt_parts, q, g2, be2, m2, v2)


def _tc_final(s_parts, cnt_parts, h, Wl, Wr, b2d):
    def body(s_ref, c_ref, h_ref, wl_ref, wr_ref, b_ref, o_ref):
        ssum = s_ref[0, :N, :] + s_ref[1, :N, :]
        cnt = c_ref[0, :N, 0:1] + c_ref[1, :N, 0:1]
        agg = ssum * (1.0 / jnp.maximum(cnt, 1.0))
        o_ref[...] = (
            jnp.dot(agg, wl_ref[...], preferred_element_type=jnp.float32)
            + jnp.dot(h_ref[...], wr_ref[...],
                      preferred_element_type=jnp.float32)
            + b_ref[...])

    return pl.pallas_call(
        body, out_shape=jax.ShapeDtypeStruct((N, D_OUT), jnp.float32),
    )(s_parts, cnt_parts, h, Wl, Wr, b2d)


def kernel(x, edge_index, Wl0, Wr0, b0, g0, be0, m0, v0,
           Wl1, Wr1, b1, g1, be1, m1, v1, Wl2, Wr2, b2):
    src = edge_index[0]
    dst = edge_index[1]
    pad = E_PAD - E
    srcs = jnp.concatenate(
        [src, jnp.zeros((pad,), jnp.int32)]).reshape(TOTAL_CH, CHUNK)
    dsts = jnp.concatenate(
        [dst, jnp.full((pad,), N, jnp.int32)]).reshape(TOTAL_CH, CHUNK)
    ones = jnp.ones((CHUNK, CNT_W), jnp.float32)

    seg_cnt = _make_sc_segsum(True)
    seg = _make_sc_segsum(False)

    r2 = lambda a: a.reshape(1, -1)

    p0, q0 = _tc_proj(x, Wl0, Wr0, r2(b0))
    s0, cnt = seg_cnt(p0, srcs, dsts, ones)
    h1 = _tc_combine(s0, cnt, q0, r2(g0), r2(be0), r2(m0), r2(v0))

    p1, q1 = _tc_proj(h1, Wl1, Wr1, r2(b1))
    outs = seg(p1, srcs, dsts)
    s1 = outs[0] if isinstance(outs, (tuple, list)) else outs
    h2 = _tc_combine(s1, cnt, q1, r2(g1), r2(be1), r2(m1), r2(v1))

    outs = seg(h2, srcs, dsts)
    s2 = outs[0] if isinstance(outs, (tuple, list)) else outs
    return _tc_final(s2, cnt, h2, Wl2, Wr2, r2(b2))
